# baseline (speedup 1.0000x reference)
import functools

import jax
import jax.numpy as jnp
from jax import lax
from jax.experimental import pallas as pl
from jax.experimental.pallas import tpu as pltpu
from jax.experimental.pallas import tpu_sc as plsc

_B, _N, _E = 8, 2048, 32768
_IN, _HID, _OUT, _L = 128, 256, 128, 3

_NS = 16
_QROWS = 512
_ACC = _QROWS * _N
_DUMP = _ACC
_EPT = _E // _NS
_TSL = _ACC // _NS
_GROUPS = _EPT // 128


_NVEC = _EPT // 16
_RING = 16


def _adj_body(edges, zeros_in, ones_in, a_out, dst_v, src_v, ones_v, zer_v,
              flush_v, acc, sem):
    c = lax.axis_index("c")
    s = lax.axis_index("s")

    pltpu.sync_copy(ones_in, ones_v)
    pltpu.sync_copy(zeros_in, zer_v)

    ebase = s * _EPT
    z0 = pl.multiple_of(s * _TSL, 8192)

    def _pass(bp, carry):
        b = bp // 2
        p = bp % 2
        row0 = (c * 2 + p) * _QROWS

        for z in range(8):
            pltpu.sync_copy(zer_v, acc.at[pl.ds(z0 + z * 8192, 8192)])
        plsc.subcore_barrier()

        pltpu.sync_copy(edges.at[b, 1, pl.ds(ebase, _EPT)], dst_v)
        pltpu.sync_copy(edges.at[b, 0, pl.ds(ebase, _EPT)], src_v)

        def _flat(j):
            d = dst_v[pl.ds(j * 16, 16)]
            sv = src_v[pl.ds(j * 16, 16)]
            rel = d - row0
            inr = (rel >= 0) & (rel < _QROWS)
            return jnp.where(inr, rel * _N + sv, _DUMP)

        cps = [None] * _NVEC
        for j in range(_NVEC):
            cps[j] = pltpu.async_copy(ones_v, acc.at[_flat(j)], sem, add=True)
            if j >= _RING:
                cps[j - _RING].wait()
        for j in range(_NVEC - _RING, _NVEC):
            cps[j].wait()
        for j in range(_NVEC):
            cps[j] = pltpu.async_copy(acc.at[_flat(j)], flush_v, sem)
            if j >= _RING:
                cps[j - _RING].wait()
        for j in range(_NVEC - _RING, _NVEC):
            cps[j].wait()
        plsc.subcore_barrier()

        off = pl.multiple_of(row0 * _N + s * _TSL, 8192)
        pltpu.sync_copy(acc.at[pl.ds(s * _TSL, _TSL)], a_out.at[b, pl.ds(off, _TSL)])
        return carry

    lax.fori_loop(0, _B * 2, _pass, 0)


@functools.cache
def _adj_kernel():
    return pl.kernel(
        _adj_body,
        out_type=jax.ShapeDtypeStruct((_B, _N * _N), jnp.float32),
        mesh=plsc.VectorSubcoreMesh(core_axis_name="c", subcore_axis_name="s"),
        scratch_types=[
            pltpu.VMEM((_EPT,), jnp.int32),
            pltpu.VMEM((_EPT,), jnp.int32),
            pltpu.VMEM((16,), jnp.float32),
            pltpu.VMEM((8192,), jnp.float32),
            pltpu.VMEM((16,), jnp.float32),
            pltpu.VMEM_SHARED((_ACC + 8,), jnp.float32),
            pltpu.SemaphoreType.DMA,
        ],
    )


def _tc_body(nf_ref, a_ref, w1_ref, b1_ref, w2_ref, b2_ref, wc_ref, bc_ref,
             g_ref, bb_ref, wo_ref, bo_ref, out_ref):
    f32 = jnp.float32
    nf = nf_ref[0]
    h = jnp.maximum(jnp.dot(nf, w1_ref[...], preferred_element_type=f32, precision=lax.Precision.HIGHEST)
                    + b1_ref[...], 0.0)
    x = jnp.dot(h, w2_ref[...], preferred_element_type=f32, precision=lax.Precision.HIGHEST) + b2_ref[...]

    a_mat = a_ref[...]
    cnt = jnp.sum(a_mat, axis=1, keepdims=True)
    den = jnp.maximum(cnt, 1.0)
    a_bf = a_mat.astype(jnp.bfloat16)

    for i in range(_L):
        wt = wc_ref[i, :_HID, :]
        wb = wc_ref[i, _HID:, :]
        hi = x.astype(jnp.bfloat16)
        lo = (x - hi.astype(f32)).astype(jnp.bfloat16)
        agg = (jnp.dot(a_bf, hi, preferred_element_type=f32)
               + jnp.dot(a_bf, lo, preferred_element_type=f32)) / den
        y = (jnp.dot(x, wt, preferred_element_type=f32, precision=lax.Precision.HIGHEST)
             + jnp.dot(agg, wb, preferred_element_type=f32, precision=lax.Precision.HIGHEST)
             + bc_ref[i] + x)
        mu = jnp.mean(y, axis=-1, keepdims=True)
        yc = y - mu
        var = jnp.mean(yc * yc, axis=-1, keepdims=True)
        x = jnp.maximum(yc / jnp.sqrt(var + 1e-5) * g_ref[i] + bb_ref[i], 0.0)

    out_ref[0] = (jnp.dot(x, wo_ref[...], preferred_element_type=f32, precision=lax.Precision.HIGHEST)
                  + bo_ref[...])


def _tc_fused(nf, a3, w1, b1, w2, b2, wc, bc, g, bb, wo, bo):
    return pl.pallas_call(
        _tc_body,
        grid=(_B,),
        in_specs=[
            pl.BlockSpec((1, _N, _IN), lambda b: (b, 0, 0)),
            pl.BlockSpec((_N, _N), lambda b: (b, 0)),
            pl.BlockSpec((_IN, _HID), lambda b: (0, 0)),
            pl.BlockSpec((1, _HID), lambda b: (0, 0)),
            pl.BlockSpec((_HID, _HID), lambda b: (0, 0)),
            pl.BlockSpec((1, _HID), lambda b: (0, 0)),
            pl.BlockSpec((_L, 2 * _HID, _HID), lambda b: (0, 0, 0)),
            pl.BlockSpec((_L, 1, _HID), lambda b: (0, 0, 0)),
            pl.BlockSpec((_L, 1, _HID), lambda b: (0, 0, 0)),
            pl.BlockSpec((_L, 1, _HID), lambda b: (0, 0, 0)),
            pl.BlockSpec((_HID, _OUT), lambda b: (0, 0)),
            pl.BlockSpec((1, _OUT), lambda b: (0, 0)),
        ],
        out_specs=pl.BlockSpec((1, _N, _OUT), lambda b: (b, 0, 0)),
        out_shape=jax.ShapeDtypeStruct((_B, _N, _OUT), jnp.float32),
        compiler_params=pltpu.CompilerParams(
            dimension_semantics=("arbitrary",)),
    )(nf, a3, w1, b1, w2, b2, wc, bc, g, bb, wo, bo)


def kernel(node_features, edge_indices, batch_size, W_enc1, b_enc1, W_enc2,
           b_enc2, W_conv, b_conv, ln_g, ln_b, W_out, b_out):
    a_flat = _adj_kernel()(edge_indices.astype(jnp.int32),
                           jnp.zeros((8192,), jnp.float32),
                           jnp.ones((16,), jnp.float32))
    a3 = a_flat.reshape(_B * _N, _N)
    return _tc_fused(
        node_features, a3,
        W_enc1, b_enc1.reshape(1, _HID),
        W_enc2, b_enc2.reshape(1, _HID),
        W_conv, b_conv.reshape(_L, 1, _HID),
        ln_g.reshape(_L, 1, _HID), ln_b.reshape(_L, 1, _HID),
        W_out, b_out.reshape(1, _OUT),
    )


# speedup vs baseline: 1.2420x; 1.2420x over previous
import functools

import jax
import jax.numpy as jnp
from jax import lax
from jax.experimental import pallas as pl
from jax.experimental.pallas import tpu as pltpu
from jax.experimental.pallas import tpu_sc as plsc

_B, _N, _E = 8, 2048, 32768
_IN, _HID, _OUT, _L = 128, 256, 128, 3

_NS = 16
_QROWS = 512
_ACC = _QROWS * _N
_DUMP = _ACC
_EPT = _E // _NS
_TSL = _ACC // _NS
_GROUPS = _EPT // 128


_NVEC = _EPT // 16
_RING = 16


def _adj_body(edges, zeros_in, ones_in, a_out, dst_v, src_v, idx2, ones_v,
              zer_v, flush_v, acc, sem):
    c = lax.axis_index("c")
    s = lax.axis_index("s")

    pltpu.sync_copy(ones_in, ones_v)
    pltpu.sync_copy(zeros_in, zer_v)

    ebase = s * _EPT
    z0 = pl.multiple_of(s * _TSL, 8192)

    def _pass(bp, carry):
        b = bp // 2
        p = bp % 2
        row0 = (c * 2 + p) * _QROWS

        for z in range(8):
            pltpu.sync_copy(zer_v, acc.at[pl.ds(z0 + z * 8192, 8192)])
        plsc.subcore_barrier()

        pltpu.sync_copy(edges.at[b, 1, pl.ds(ebase, _EPT)], dst_v)
        pltpu.sync_copy(edges.at[b, 0, pl.ds(ebase, _EPT)], src_v)

        for g in range(_GROUPS):
            def _vec(j, carry2, g=g):
                e0 = g * 128 + j * 16
                d = dst_v[pl.ds(e0, 16)]
                sv = src_v[pl.ds(e0, 16)]
                rel = d - row0
                inr = (rel >= 0) & (rel < _QROWS)
                idx2[g, pl.ds(j * 16, 16)] = jnp.where(inr, rel * _N + sv, _DUMP)
                return carry2
            lax.fori_loop(0, 128 // 16, _vec, 0)

        cps = [pltpu.async_copy(ones_v, acc.at[idx2.at[g]], sem, add=True)
               for g in range(_GROUPS)]
        for cp in cps:
            cp.wait()
        gps = [pltpu.async_copy(acc.at[idx2.at[g]], flush_v, sem)
               for g in range(_GROUPS)]
        for cp in gps:
            cp.wait()
        plsc.subcore_barrier()

        off = pl.multiple_of(row0 * _N + s * _TSL, 8192)
        pltpu.sync_copy(acc.at[pl.ds(s * _TSL, _TSL)], a_out.at[b, pl.ds(off, _TSL)])
        return carry

    lax.fori_loop(0, _B * 2, _pass, 0)


@functools.cache
def _adj_kernel():
    return pl.kernel(
        _adj_body,
        out_type=jax.ShapeDtypeStruct((_B, _N * _N), jnp.float32),
        mesh=plsc.VectorSubcoreMesh(core_axis_name="c", subcore_axis_name="s"),
        scratch_types=[
            pltpu.VMEM((_EPT,), jnp.int32),
            pltpu.VMEM((_EPT,), jnp.int32),
            pltpu.VMEM((_GROUPS, 128), jnp.int32),
            pltpu.VMEM((128,), jnp.float32),
            pltpu.VMEM((8192,), jnp.float32),
            pltpu.VMEM((128,), jnp.float32),
            pltpu.VMEM_SHARED((_ACC + 8,), jnp.float32),
            pltpu.SemaphoreType.DMA,
        ],
    )


def _tc_body(nf_ref, a_ref, w1_ref, b1_ref, w2_ref, b2_ref, wc_ref, bc_ref,
             g_ref, bb_ref, wo_ref, bo_ref, out_ref):
    f32 = jnp.float32
    nf = nf_ref[0]
    h = jnp.maximum(jnp.dot(nf, w1_ref[...], preferred_element_type=f32, precision=lax.Precision.HIGHEST)
                    + b1_ref[...], 0.0)
    x = jnp.dot(h, w2_ref[...], preferred_element_type=f32, precision=lax.Precision.HIGHEST) + b2_ref[...]

    a_mat = a_ref[0]
    cnt = jnp.sum(a_mat, axis=1, keepdims=True)
    den = jnp.maximum(cnt, 1.0)
    a_bf = a_mat.astype(jnp.bfloat16)

    for i in range(_L):
        wt = wc_ref[i, :_HID, :]
        wb = wc_ref[i, _HID:, :]
        hi = x.astype(jnp.bfloat16)
        lo = (x - hi.astype(f32)).astype(jnp.bfloat16)
        agg = (jnp.dot(a_bf, hi, preferred_element_type=f32)
               + jnp.dot(a_bf, lo, preferred_element_type=f32)) / den
        y = (jnp.dot(x, wt, preferred_element_type=f32, precision=lax.Precision.HIGHEST)
             + jnp.dot(agg, wb, preferred_element_type=f32, precision=lax.Precision.HIGHEST)
             + bc_ref[i] + x)
        mu = jnp.mean(y, axis=-1, keepdims=True)
        yc = y - mu
        var = jnp.mean(yc * yc, axis=-1, keepdims=True)
        x = jnp.maximum(yc / jnp.sqrt(var + 1e-5) * g_ref[i] + bb_ref[i], 0.0)

    out_ref[0] = (jnp.dot(x, wo_ref[...], preferred_element_type=f32, precision=lax.Precision.HIGHEST)
                  + bo_ref[...])


def _tc_fused(nf, a3, w1, b1, w2, b2, wc, bc, g, bb, wo, bo):
    return pl.pallas_call(
        _tc_body,
        grid=(_B,),
        in_specs=[
            pl.BlockSpec((1, _N, _IN), lambda b: (b, 0, 0)),
            pl.BlockSpec((1, _N, _N), lambda b: (b, 0, 0)),
            pl.BlockSpec((_IN, _HID), lambda b: (0, 0)),
            pl.BlockSpec((1, _HID), lambda b: (0, 0)),
            pl.BlockSpec((_HID, _HID), lambda b: (0, 0)),
            pl.BlockSpec((1, _HID), lambda b: (0, 0)),
            pl.BlockSpec((_L, 2 * _HID, _HID), lambda b: (0, 0, 0)),
            pl.BlockSpec((_L, 1, _HID), lambda b: (0, 0, 0)),
            pl.BlockSpec((_L, 1, _HID), lambda b: (0, 0, 0)),
            pl.BlockSpec((_L, 1, _HID), lambda b: (0, 0, 0)),
            pl.BlockSpec((_HID, _OUT), lambda b: (0, 0)),
            pl.BlockSpec((1, _OUT), lambda b: (0, 0)),
        ],
        out_specs=pl.BlockSpec((1, _N, _OUT), lambda b: (b, 0, 0)),
        out_shape=jax.ShapeDtypeStruct((_B, _N, _OUT), jnp.float32),
        compiler_params=pltpu.CompilerParams(
            dimension_semantics=("arbitrary",)),
    )(nf, a3, w1, b1, w2, b2, wc, bc, g, bb, wo, bo)


def kernel(node_features, edge_indices, batch_size, W_enc1, b_enc1, W_enc2,
           b_enc2, W_conv, b_conv, ln_g, ln_b, W_out, b_out):
    a_flat = _adj_kernel()(edge_indices.astype(jnp.int32),
                           jnp.zeros((8192,), jnp.float32),
                           jnp.ones((128,), jnp.float32))
    a3 = a_flat.reshape(_B, _N, _N)
    return _tc_fused(
        node_features, a3,
        W_enc1, b_enc1.reshape(1, _HID),
        W_enc2, b_enc2.reshape(1, _HID),
        W_conv, b_conv.reshape(_L, 1, _HID),
        ln_g.reshape(_L, 1, _HID), ln_b.reshape(_L, 1, _HID),
        W_out, b_out.reshape(1, _OUT),
    )


# speedup vs baseline: 1.2578x; 1.0127x over previous
import functools

import jax
import jax.numpy as jnp
from jax import lax
from jax.experimental import pallas as pl
from jax.experimental.pallas import tpu as pltpu
from jax.experimental.pallas import tpu_sc as plsc

_B, _N, _E = 8, 2048, 32768
_IN, _HID, _OUT, _L = 128, 256, 128, 3

_NS = 16
_QROWS = 512
_ACC = _QROWS * _N
_DUMP = _ACC
_EPT = _E // _NS
_TSL = _ACC // _NS
_GROUPS = _EPT // 128


_ZRV = 32768
_NVEC = _EPT // 16
_RING = 16


def _adj_body(edges, zeros_in, ones_in, a_out, dst_v, src_v, idx2, ones_v,
              zer_v, flush_v, acc, sem, zsem, esem):
    c = lax.axis_index("c")
    s = lax.axis_index("s")

    pltpu.sync_copy(ones_in, ones_v)
    for z in range(_ZRV // 8192):
        pltpu.sync_copy(zeros_in, zer_v.at[pl.ds(z * 8192, 8192)])

    ebase = s * _EPT
    z0 = pl.multiple_of(s * _TSL, 8192)

    def _pass(bp, carry):
        b = bp // 2
        p = bp % 2
        row0 = (c * 2 + p) * _QROWS

        zcps = [pltpu.async_copy(zer_v, acc.at[pl.ds(z0 + z * _ZRV, _ZRV)], zsem)
                for z in range(_TSL // _ZRV)]
        e1 = pltpu.async_copy(edges.at[b, 1, pl.ds(ebase, _EPT)], dst_v, esem)
        e2 = pltpu.async_copy(edges.at[b, 0, pl.ds(ebase, _EPT)], src_v, esem)
        e1.wait()
        e2.wait()

        for g in range(_GROUPS):
            def _vec(j, carry2, g=g):
                e0 = g * 128 + j * 16
                d = dst_v[pl.ds(e0, 16)]
                sv = src_v[pl.ds(e0, 16)]
                rel = d - row0
                inr = (rel >= 0) & (rel < _QROWS)
                idx2[g, pl.ds(j * 16, 16)] = jnp.where(inr, rel * _N + sv, _DUMP)
                return carry2
            lax.fori_loop(0, 128 // 16, _vec, 0)
        for zcp in zcps:
            zcp.wait()
        plsc.subcore_barrier()

        cps = [pltpu.async_copy(ones_v, acc.at[idx2.at[g]], sem, add=True)
               for g in range(_GROUPS)]
        for cp in cps:
            cp.wait()
        gps = [pltpu.async_copy(acc.at[idx2.at[g]], flush_v, sem)
               for g in range(_GROUPS)]
        for cp in gps:
            cp.wait()
        plsc.subcore_barrier()

        off = pl.multiple_of(row0 * _N + s * _TSL, 8192)
        pltpu.sync_copy(acc.at[pl.ds(s * _TSL, _TSL)], a_out.at[b, pl.ds(off, _TSL)])
        return carry

    lax.fori_loop(0, _B * 2, _pass, 0)


@functools.cache
def _adj_kernel():
    return pl.kernel(
        _adj_body,
        out_type=jax.ShapeDtypeStruct((_B, _N * _N), jnp.float32),
        mesh=plsc.VectorSubcoreMesh(core_axis_name="c", subcore_axis_name="s"),
        scratch_types=[
            pltpu.VMEM((_EPT,), jnp.int32),
            pltpu.VMEM((_EPT,), jnp.int32),
            pltpu.VMEM((_GROUPS, 128), jnp.int32),
            pltpu.VMEM((128,), jnp.float32),
            pltpu.VMEM((_ZRV,), jnp.float32),
            pltpu.VMEM((128,), jnp.float32),
            pltpu.VMEM_SHARED((_ACC + 8,), jnp.float32),
            pltpu.SemaphoreType.DMA,
            pltpu.SemaphoreType.DMA,
            pltpu.SemaphoreType.DMA,
        ],
    )


def _tc_body(nf_ref, a_ref, w1_ref, b1_ref, w2_ref, b2_ref, wc_ref, bc_ref,
             g_ref, bb_ref, wo_ref, bo_ref, out_ref):
    f32 = jnp.float32
    nf = nf_ref[0]
    h = jnp.maximum(jnp.dot(nf, w1_ref[...], preferred_element_type=f32, precision=lax.Precision.HIGHEST)
                    + b1_ref[...], 0.0)
    x = jnp.dot(h, w2_ref[...], preferred_element_type=f32, precision=lax.Precision.HIGHEST) + b2_ref[...]

    a_mat = a_ref[0]
    cnt = jnp.sum(a_mat, axis=1, keepdims=True)
    den = jnp.maximum(cnt, 1.0)
    a_bf = a_mat.astype(jnp.bfloat16)

    for i in range(_L):
        wt = wc_ref[i, :_HID, :]
        wb = wc_ref[i, _HID:, :]
        hi = x.astype(jnp.bfloat16)
        lo = (x - hi.astype(f32)).astype(jnp.bfloat16)
        agg = (jnp.dot(a_bf, hi, preferred_element_type=f32)
               + jnp.dot(a_bf, lo, preferred_element_type=f32)) / den
        y = (jnp.dot(x, wt, preferred_element_type=f32, precision=lax.Precision.HIGHEST)
             + jnp.dot(agg, wb, preferred_element_type=f32, precision=lax.Precision.HIGHEST)
             + bc_ref[i] + x)
        mu = jnp.mean(y, axis=-1, keepdims=True)
        yc = y - mu
        var = jnp.mean(yc * yc, axis=-1, keepdims=True)
        x = jnp.maximum(yc / jnp.sqrt(var + 1e-5) * g_ref[i] + bb_ref[i], 0.0)

    out_ref[0] = (jnp.dot(x, wo_ref[...], preferred_element_type=f32, precision=lax.Precision.HIGHEST)
                  + bo_ref[...])


def _tc_fused(nf, a3, w1, b1, w2, b2, wc, bc, g, bb, wo, bo):
    return pl.pallas_call(
        _tc_body,
        grid=(_B,),
        in_specs=[
            pl.BlockSpec((1, _N, _IN), lambda b: (b, 0, 0)),
            pl.BlockSpec((1, _N, _N), lambda b: (b, 0, 0)),
            pl.BlockSpec((_IN, _HID), lambda b: (0, 0)),
            pl.BlockSpec((1, _HID), lambda b: (0, 0)),
            pl.BlockSpec((_HID, _HID), lambda b: (0, 0)),
            pl.BlockSpec((1, _HID), lambda b: (0, 0)),
            pl.BlockSpec((_L, 2 * _HID, _HID), lambda b: (0, 0, 0)),
            pl.BlockSpec((_L, 1, _HID), lambda b: (0, 0, 0)),
            pl.BlockSpec((_L, 1, _HID), lambda b: (0, 0, 0)),
            pl.BlockSpec((_L, 1, _HID), lambda b: (0, 0, 0)),
            pl.BlockSpec((_HID, _OUT), lambda b: (0, 0)),
            pl.BlockSpec((1, _OUT), lambda b: (0, 0)),
        ],
        out_specs=pl.BlockSpec((1, _N, _OUT), lambda b: (b, 0, 0)),
        out_shape=jax.ShapeDtypeStruct((_B, _N, _OUT), jnp.float32),
        compiler_params=pltpu.CompilerParams(
            dimension_semantics=("arbitrary",)),
    )(nf, a3, w1, b1, w2, b2, wc, bc, g, bb, wo, bo)


def kernel(node_features, edge_indices, batch_size, W_enc1, b_enc1, W_enc2,
           b_enc2, W_conv, b_conv, ln_g, ln_b, W_out, b_out):
    a_flat = _adj_kernel()(edge_indices.astype(jnp.int32),
                           jnp.zeros((8192,), jnp.float32),
                           jnp.ones((128,), jnp.float32))
    a3 = a_flat.reshape(_B, _N, _N)
    return _tc_fused(
        node_features, a3,
        W_enc1, b_enc1.reshape(1, _HID),
        W_enc2, b_enc2.reshape(1, _HID),
        W_conv, b_conv.reshape(_L, 1, _HID),
        ln_g.reshape(_L, 1, _HID), ln_b.reshape(_L, 1, _HID),
        W_out, b_out.reshape(1, _OUT),
    )
